# baseline (speedup 1.0000x reference)
import functools

import jax
import jax.numpy as jnp
from jax import lax
from jax.experimental import pallas as pl
from jax.experimental.pallas import tpu as pltpu
from jax.experimental.pallas import tpu_sc as plsc

_S = 8
_DSUB = 32
_K = 256
_NBLK = 4096


def _enc_body(x_ref, cb_ref, lab_ref):
    for s in range(_S):
        xs = x_ref[s * _DSUB:(s + 1) * _DSUB, :]
        cb = cb_ref[s]
        c2 = jnp.sum(cb * cb, axis=0)
        prod = jax.lax.dot_general(
            cb * -2.0, xs, (((0,), (0,)), ((), ())),
            preferred_element_type=jnp.float32)
        scores = prod + c2[:, None]
        minval = jnp.min(scores, axis=0)
        mask = scores == minval[None, :]
        ki = jnp.where(mask,
                       jax.lax.broadcasted_iota(jnp.int32, scores.shape, 0),
                       _K)
        lab_ref[s, :] = jnp.min(ki, axis=0)


def _tc_encode(x, codebook):
    D, N = x.shape
    grid = (N // _NBLK,)
    return pl.pallas_call(
        _enc_body,
        grid=grid,
        in_specs=[
            pl.BlockSpec((D, _NBLK), lambda i: (0, i)),
            pl.BlockSpec((_S, _DSUB, _K), lambda i: (0, 0, 0)),
        ],
        out_specs=pl.BlockSpec((_S, _NBLK), lambda i: (0, i)),
        out_shape=jax.ShapeDtypeStruct((_S, N), jnp.int32),
    )(x, codebook)


def _sc_decode(labels, codebook, n):
    info = plsc.get_sparse_core_info()
    nc, ns, nl = info.num_cores, info.num_subcores, info.num_lanes
    nw = nc * ns
    cper = n // nw
    labels3 = labels.reshape(_S, nw, cper)
    cb2 = codebook.reshape(_S, _DSUB * _K)
    mesh = plsc.VectorSubcoreMesh(core_axis_name="c", subcore_axis_name="s")

    @functools.partial(
        pl.kernel, mesh=mesh,
        out_type=jax.ShapeDtypeStruct((_S, _DSUB, nw, cper), jnp.float32),
        scratch_types=[
            pltpu.VMEM((cper,), jnp.int32),
            pltpu.VMEM((_DSUB * _K,), jnp.float32),
            pltpu.VMEM((_DSUB * cper,), jnp.float32),
            pltpu.SemaphoreType.DMA,
        ],
        compiler_params=pltpu.CompilerParams(needs_layout_passes=False),
    )
    def k(lab_hbm, cb_hbm, out_hbm, lab_v, cb_v, strip_v, sem):
        wid = lax.axis_index("s") * nc + lax.axis_index("c")
        for s in range(_S):
            pltpu.sync_copy(lab_hbm.at[s, wid], lab_v)
            pltpu.sync_copy(cb_hbm.at[s], cb_v)
            for d in range(_DSUB):

                def body(j, _, d=d):
                    lab16 = lab_v[pl.ds(j * nl, nl)]
                    vals = plsc.load_gather(cb_v, [lab16 + d * _K])
                    strip_v[pl.ds(d * cper + j * nl, nl)] = vals
                    return 0

                lax.fori_loop(0, cper // nl, body, 0)
            for d in range(_DSUB):
                pltpu.async_copy(strip_v.at[pl.ds(d * cper, cper)],
                                 out_hbm.at[s, d, wid], sem)
            for d in range(_DSUB):
                pltpu.make_async_copy(strip_v.at[pl.ds(d * cper, cper)],
                                      out_hbm.at[s, d, wid], sem).wait()

    out4 = k(labels3, cb2)
    return out4


@functools.partial(jax.jit, static_argnames=())
def kernel(x, codebook):
    D, N = x.shape
    labels = _tc_encode(x, codebook)
    out4 = _sc_decode(labels, codebook, N)
    return out4.reshape(D, N)


# speedup vs baseline: 4.7098x; 4.7098x over previous
import functools

import jax
import jax.numpy as jnp
from jax.experimental import pallas as pl

_S = 8
_DSUB = 32
_K = 256
_NBLK = 4096


def _pq_body(x_ref, cb_ref, cbm2_ref, c2_ref, out_ref):
    for s in range(_S):
        xs = x_ref[s * _DSUB:(s + 1) * _DSUB, :]
        prod = jax.lax.dot_general(
            cbm2_ref[s], xs, (((0,), (0,)), ((), ())),
            preferred_element_type=jnp.float32)
        bias = jax.lax.dot_general(
            c2_ref[s][:, None], jnp.ones((1, _NBLK), jnp.float32),
            (((1,), (0,)), ((), ())),
            preferred_element_type=jnp.float32)
        scores = prod + bias
        minval = jnp.min(scores, axis=0)
        onehot = (scores == minval[None, :]).astype(jnp.float32)
        out_ref[s * _DSUB:(s + 1) * _DSUB, :] = jax.lax.dot_general(
            cb_ref[s], onehot, (((1,), (0,)), ((), ())),
            preferred_element_type=jnp.float32)


@functools.partial(jax.jit, static_argnames=())
def kernel(x, codebook):
    D, N = x.shape
    cbm2 = codebook * -2.0
    c2 = jnp.sum(codebook * codebook, axis=1)
    grid = (N // _NBLK,)
    return pl.pallas_call(
        _pq_body,
        grid=grid,
        in_specs=[
            pl.BlockSpec((D, _NBLK), lambda i: (0, i)),
            pl.BlockSpec((_S, _DSUB, _K), lambda i: (0, 0, 0)),
            pl.BlockSpec((_S, _DSUB, _K), lambda i: (0, 0, 0)),
            pl.BlockSpec((_S, _K), lambda i: (0, 0)),
        ],
        out_specs=pl.BlockSpec((D, _NBLK), lambda i: (0, i)),
        out_shape=jax.ShapeDtypeStruct((D, N), jnp.float32),
    )(x, codebook, cbm2, c2)
